# Initial kernel scaffold; baseline (speedup 1.0000x reference)
import functools

import jax
import jax.numpy as jnp
from jax import lax
from jax.experimental import pallas as pl
from jax.experimental.pallas import tpu as pltpu
from jax.experimental.pallas import tpu_sc as plsc

NC = 2
NS = 16
NW = NC * NS
CH = 128
BR = 1024


def _sc_mesh():
    return plsc.VectorSubcoreMesh(
        core_axis_name="c", subcore_axis_name="s", num_cores=NC, num_subcores=NS
    )


def _deg_partials(dst_p, ones16, zeros16, npad, ept, nch):
    @functools.partial(
        pl.kernel,
        out_type=jax.ShapeDtypeStruct((NC, npad, 16), jnp.float32),
        mesh=_sc_mesh(),
        scratch_types=[
            pltpu.VMEM((CH,), jnp.int32),
            pltpu.VMEM((CH, 16), jnp.float32),
            pltpu.VMEM_SHARED((npad, 16), jnp.float32),
        ],
    )
    def deg_kernel(dst_hbm, ones_hbm, zeros_hbm, out_hbm, didx, ones_v, acc):
        c = lax.axis_index("c")
        s = lax.axis_index("s")
        rpt = npad // NS
        pltpu.sync_copy(zeros_hbm.at[pl.ds(s * rpt, rpt)], acc.at[pl.ds(s * rpt, rpt)])
        pltpu.sync_copy(ones_hbm, ones_v)
        plsc.subcore_barrier()
        base = (c * NS + s) * ept

        @pl.loop(0, nch)
        def _(j):
            pltpu.sync_copy(dst_hbm.at[pl.ds(base + j * CH, CH)], didx)
            pltpu.sync_copy(ones_v, acc.at[didx], add=True)

        plsc.subcore_barrier()
        pltpu.sync_copy(acc.at[pl.ds(s * rpt, rpt)], out_hbm.at[c, pl.ds(s * rpt, rpt)])

    return deg_kernel(dst_p, ones16, zeros16)


def _scatter_partials(z, src_p, dst_p, zeros_row, npad, ept, nch):
    @functools.partial(
        pl.kernel,
        out_type=jax.ShapeDtypeStruct((NC, npad, 128), jnp.float32),
        mesh=_sc_mesh(),
        scratch_types=[
            pltpu.VMEM((CH,), jnp.int32),
            pltpu.VMEM((CH,), jnp.int32),
            pltpu.VMEM((CH, 128), jnp.float32),
            pltpu.VMEM_SHARED((npad, 128), jnp.float32),
        ],
    )
    def scat_kernel(z_hbm, src_hbm, dst_hbm, zeros_hbm, out_hbm, sidx, didx, rows, acc):
        c = lax.axis_index("c")
        s = lax.axis_index("s")
        rpt = npad // NS
        pltpu.sync_copy(zeros_hbm.at[pl.ds(s * rpt, rpt)], acc.at[pl.ds(s * rpt, rpt)])
        plsc.subcore_barrier()
        base = (c * NS + s) * ept

        @pl.loop(0, nch)
        def _(j):
            eb = base + j * CH
            pltpu.sync_copy(src_hbm.at[pl.ds(eb, CH)], sidx)
            pltpu.sync_copy(dst_hbm.at[pl.ds(eb, CH)], didx)
            pltpu.sync_copy(z_hbm.at[sidx], rows)
            pltpu.sync_copy(rows, acc.at[didx], add=True)

        plsc.subcore_barrier()
        pltpu.sync_copy(acc.at[pl.ds(s * rpt, rpt)], out_hbm.at[c, pl.ds(s * rpt, rpt)])

    return scat_kernel(z, src_p, dst_p, zeros_row)


def _row_spec():
    return pl.BlockSpec((BR, 128), lambda i: (i, 0))


def _w_spec():
    return pl.BlockSpec((128, 128), lambda i: (0, 0))


def _b_spec():
    return pl.BlockSpec((1, 128), lambda i: (0, 0))


def _s_spec():
    return pl.BlockSpec((2, BR, 128), lambda i: (0, i, 0))


def _dinv_from_deg(degp, npad):
    def body(dp_ref, dv_ref):
        dp = dp_ref[...]
        deg = 1.0 + dp[0, :, :1] + dp[1, :, :1]
        dv_ref[...] = jnp.broadcast_to(lax.rsqrt(deg), dv_ref.shape)

    return pl.pallas_call(
        body,
        out_shape=jax.ShapeDtypeStruct((npad, 128), jnp.float32),
    )(degp)


def _z_proj(act, w, dinv):
    npad = act.shape[0]

    def body(a_ref, w_ref, dv_ref, z_ref):
        z_ref[...] = dv_ref[...] * jnp.dot(
            a_ref[...], w_ref[...], preferred_element_type=jnp.float32
        )

    return pl.pallas_call(
        body,
        grid=(npad // BR,),
        in_specs=[_row_spec(), _w_spec(), _row_spec()],
        out_specs=_row_spec(),
        out_shape=jax.ShapeDtypeStruct((npad, 128), jnp.float32),
    )(act, w, dinv)


def _linear(act, w, b):
    npad = act.shape[0]

    def body(a_ref, w_ref, b_ref, o_ref):
        o_ref[...] = (
            jnp.dot(a_ref[...], w_ref[...], preferred_element_type=jnp.float32)
            + b_ref[...]
        )

    return pl.pallas_call(
        body,
        grid=(npad // BR,),
        in_specs=[_row_spec(), _w_spec(), _b_spec()],
        out_specs=_row_spec(),
        out_shape=jax.ShapeDtypeStruct((npad, 128), jnp.float32),
    )(act, w, b)


def _stage_b(s, z0, dinv, b, w):
    npad = z0.shape[0]

    def body(s_ref, z0_ref, dv_ref, b_ref, w_ref, h_ref, z1_ref):
        dv = dv_ref[...]
        g = dv * (s_ref[0] + s_ref[1] + z0_ref[...]) + b_ref[...]
        h = jnp.maximum(g, 0.0)
        h_ref[...] = h
        z1_ref[...] = dv * jnp.dot(h, w_ref[...], preferred_element_type=jnp.float32)

    return pl.pallas_call(
        body,
        grid=(npad // BR,),
        in_specs=[_s_spec(), _row_spec(), _row_spec(), _b_spec(), _w_spec()],
        out_specs=[_row_spec(), _row_spec()],
        out_shape=[
            jax.ShapeDtypeStruct((npad, 128), jnp.float32),
            jax.ShapeDtypeStruct((npad, 128), jnp.float32),
        ],
    )(s, z0, dinv, b, w)


def _stage_c(s, z1, dinv, b, skip, w):
    npad = z1.shape[0]

    def body(s_ref, z1_ref, dv_ref, b_ref, k_ref, w_ref, z2_ref):
        dv = dv_ref[...]
        g = dv * (s_ref[0] + s_ref[1] + z1_ref[...]) + b_ref[...]
        h = jnp.maximum(g, 0.0) + k_ref[...]
        z2_ref[...] = dv * jnp.dot(h, w_ref[...], preferred_element_type=jnp.float32)

    return pl.pallas_call(
        body,
        grid=(npad // BR,),
        in_specs=[_s_spec(), _row_spec(), _row_spec(), _b_spec(), _row_spec(), _w_spec()],
        out_specs=_row_spec(),
        out_shape=jax.ShapeDtypeStruct((npad, 128), jnp.float32),
    )(s, z1, dinv, b, skip, w)


def _stage_d(s, z2, dinv, b):
    npad = z2.shape[0]

    def body(s_ref, z2_ref, dv_ref, b_ref, o_ref):
        o_ref[...] = dv_ref[...] * (s_ref[0] + s_ref[1] + z2_ref[...]) + b_ref[...]

    return pl.pallas_call(
        body,
        grid=(npad // BR,),
        in_specs=[_s_spec(), _row_spec(), _row_spec(), _b_spec()],
        out_specs=_row_spec(),
        out_shape=jax.ShapeDtypeStruct((npad, 128), jnp.float32),
    )(s, z2, dinv, b)


def kernel(x, edge_index, W0, b0, W1, b1, W2, b2, Ws, bs):
    n, d_in = x.shape
    e = edge_index.shape[1]
    assert d_in == 128

    npad = -(-(n + 1) // BR) * BR
    ept = -(-e // (NW * CH)) * CH
    epad = ept * NW
    nch = ept // CH

    src = edge_index[0]
    dst = edge_index[1]
    pad_e = epad - e
    src_p = jnp.concatenate([src, jnp.zeros((pad_e,), jnp.int32)])
    dst_p = jnp.concatenate([dst, jnp.full((pad_e,), n, jnp.int32)])
    x_p = jnp.zeros((npad, d_in), jnp.float32).at[:n].set(x)
    zeros16 = jnp.zeros((npad, 16), jnp.float32)
    zeros_row = jnp.zeros((npad, 128), jnp.float32)
    ones16 = jnp.ones((CH, 16), jnp.float32)
    b0r = b0.reshape(1, 128)
    b1r = b1.reshape(1, 128)
    b2r = b2.reshape(1, 128)
    bsr = bs.reshape(1, 128)

    degp = _deg_partials(dst_p, ones16, zeros16, npad, ept, nch)
    dinv = _dinv_from_deg(degp, npad)

    z0 = _z_proj(x_p, W0, dinv)
    s0 = _scatter_partials(z0, src_p, dst_p, zeros_row, npad, ept, nch)
    h0, z1 = _stage_b(s0, z0, dinv, b0r, W1)
    skip = _linear(h0, Ws, bsr)
    s1 = _scatter_partials(z1, src_p, dst_p, zeros_row, npad, ept, nch)
    z2 = _stage_c(s1, z1, dinv, b1r, skip, W2)
    s2 = _scatter_partials(z2, src_p, dst_p, zeros_row, npad, ept, nch)
    out = _stage_d(s2, z2, dinv, b2r)
    return out[:n]


# speedup vs baseline: 9.2134x; 9.2134x over previous
import dataclasses
import functools

import jax
import jax.numpy as jnp
from jax import lax
from jax.experimental import pallas as pl
from jax.experimental.pallas import tpu as pltpu
from jax.experimental.pallas import tpu_sc as plsc

NC = 2
NS = 16
NW = NC * NS
CH = 128
BR = 1024


def _sc_mesh():
    return plsc.VectorSubcoreMesh(
        core_axis_name="c", subcore_axis_name="s", num_cores=NC, num_subcores=NS
    )


def _deg_partials(dst_p, zeros1d, npad, ept, nch):
    @functools.partial(
        pl.kernel,
        out_type=jax.ShapeDtypeStruct((NW, npad), jnp.float32),
        mesh=_sc_mesh(),
        scratch_types=[
            pltpu.VMEM((CH,), jnp.int32),
            pltpu.VMEM((npad,), jnp.float32),
        ],
        compiler_params=dataclasses.replace(
            pltpu.CompilerParams(), needs_layout_passes=False
        ),
    )
    def deg_kernel(dst_hbm, zeros_hbm, out_hbm, didx, hist):
        c = lax.axis_index("c")
        s = lax.axis_index("s")
        w = c * NS + s
        pltpu.sync_copy(zeros_hbm, hist)
        base = w * ept

        @pl.loop(0, nch)
        def _(j):
            pltpu.sync_copy(dst_hbm.at[pl.ds(base + j * CH, CH)], didx)

            @pl.loop(0, CH // 16)
            def __(k):
                idxv = didx[pl.ds(k * 16, 16)]
                plsc.addupdate_scatter(hist, [idxv], jnp.full((16,), 1.0, jnp.float32))

        pltpu.sync_copy(hist, out_hbm.at[w])

    return deg_kernel(dst_p, zeros1d)


def _scatter_partials(z, src_p, dst_p, zeros_row, npad, ept, nch):
    @functools.partial(
        pl.kernel,
        out_type=jax.ShapeDtypeStruct((NC, npad, 128), jnp.float32),
        mesh=_sc_mesh(),
        scratch_types=[
            pltpu.VMEM((CH,), jnp.int32),
            pltpu.VMEM((CH,), jnp.int32),
            pltpu.VMEM((CH, 128), jnp.float32),
            pltpu.VMEM_SHARED((npad, 128), jnp.float32),
        ],
    )
    def scat_kernel(z_hbm, src_hbm, dst_hbm, zeros_hbm, out_hbm, sidx, didx, rows, acc):
        c = lax.axis_index("c")
        s = lax.axis_index("s")
        rpt = npad // NS
        pltpu.sync_copy(zeros_hbm.at[pl.ds(s * rpt, rpt)], acc.at[pl.ds(s * rpt, rpt)])
        plsc.subcore_barrier()
        base = (c * NS + s) * ept

        @pl.loop(0, nch)
        def _(j):
            eb = base + j * CH
            pltpu.sync_copy(src_hbm.at[pl.ds(eb, CH)], sidx)
            pltpu.sync_copy(dst_hbm.at[pl.ds(eb, CH)], didx)
            pltpu.sync_copy(z_hbm.at[sidx], rows)
            pltpu.sync_copy(rows, acc.at[didx], add=True)

        plsc.subcore_barrier()
        pltpu.sync_copy(acc.at[pl.ds(s * rpt, rpt)], out_hbm.at[c, pl.ds(s * rpt, rpt)])

    return scat_kernel(z, src_p, dst_p, zeros_row)


def _row_spec():
    return pl.BlockSpec((BR, 128), lambda i: (i, 0))


def _w_spec():
    return pl.BlockSpec((128, 128), lambda i: (0, 0))


def _b_spec():
    return pl.BlockSpec((1, 128), lambda i: (0, 0))


def _s_spec():
    return pl.BlockSpec((2, BR, 128), lambda i: (0, i, 0))


def _dinv_from_deg(degp, npad):
    def body(dp_ref, dv_ref):
        dp = dp_ref[...]
        ones = jnp.ones((NW, 128), jnp.float32)
        deg = 1.0 + lax.dot_general(
            dp, ones, (((0,), (0,)), ((), ())), preferred_element_type=jnp.float32
        )
        dv_ref[...] = lax.rsqrt(deg)

    return pl.pallas_call(
        body,
        out_shape=jax.ShapeDtypeStruct((npad, 128), jnp.float32),
    )(degp)


def _z_proj(act, w, dinv):
    npad = act.shape[0]

    def body(a_ref, w_ref, dv_ref, z_ref):
        z_ref[...] = dv_ref[...] * jnp.dot(
            a_ref[...], w_ref[...], preferred_element_type=jnp.float32
        )

    return pl.pallas_call(
        body,
        grid=(npad // BR,),
        in_specs=[_row_spec(), _w_spec(), _row_spec()],
        out_specs=_row_spec(),
        out_shape=jax.ShapeDtypeStruct((npad, 128), jnp.float32),
    )(act, w, dinv)


def _linear(act, w, b):
    npad = act.shape[0]

    def body(a_ref, w_ref, b_ref, o_ref):
        o_ref[...] = (
            jnp.dot(a_ref[...], w_ref[...], preferred_element_type=jnp.float32)
            + b_ref[...]
        )

    return pl.pallas_call(
        body,
        grid=(npad // BR,),
        in_specs=[_row_spec(), _w_spec(), _b_spec()],
        out_specs=_row_spec(),
        out_shape=jax.ShapeDtypeStruct((npad, 128), jnp.float32),
    )(act, w, b)


def _stage_b(s, z0, dinv, b, w):
    npad = z0.shape[0]

    def body(s_ref, z0_ref, dv_ref, b_ref, w_ref, h_ref, z1_ref):
        dv = dv_ref[...]
        g = dv * (s_ref[0] + s_ref[1] + z0_ref[...]) + b_ref[...]
        h = jnp.maximum(g, 0.0)
        h_ref[...] = h
        z1_ref[...] = dv * jnp.dot(h, w_ref[...], preferred_element_type=jnp.float32)

    return pl.pallas_call(
        body,
        grid=(npad // BR,),
        in_specs=[_s_spec(), _row_spec(), _row_spec(), _b_spec(), _w_spec()],
        out_specs=[_row_spec(), _row_spec()],
        out_shape=[
            jax.ShapeDtypeStruct((npad, 128), jnp.float32),
            jax.ShapeDtypeStruct((npad, 128), jnp.float32),
        ],
    )(s, z0, dinv, b, w)


def _stage_c(s, z1, dinv, b, skip, w):
    npad = z1.shape[0]

    def body(s_ref, z1_ref, dv_ref, b_ref, k_ref, w_ref, z2_ref):
        dv = dv_ref[...]
        g = dv * (s_ref[0] + s_ref[1] + z1_ref[...]) + b_ref[...]
        h = jnp.maximum(g, 0.0) + k_ref[...]
        z2_ref[...] = dv * jnp.dot(h, w_ref[...], preferred_element_type=jnp.float32)

    return pl.pallas_call(
        body,
        grid=(npad // BR,),
        in_specs=[_s_spec(), _row_spec(), _row_spec(), _b_spec(), _row_spec(), _w_spec()],
        out_specs=_row_spec(),
        out_shape=jax.ShapeDtypeStruct((npad, 128), jnp.float32),
    )(s, z1, dinv, b, skip, w)


def _stage_d(s, z2, dinv, b):
    npad = z2.shape[0]

    def body(s_ref, z2_ref, dv_ref, b_ref, o_ref):
        o_ref[...] = dv_ref[...] * (s_ref[0] + s_ref[1] + z2_ref[...]) + b_ref[...]

    return pl.pallas_call(
        body,
        grid=(npad // BR,),
        in_specs=[_s_spec(), _row_spec(), _row_spec(), _b_spec()],
        out_specs=_row_spec(),
        out_shape=jax.ShapeDtypeStruct((npad, 128), jnp.float32),
    )(s, z2, dinv, b)


def kernel(x, edge_index, W0, b0, W1, b1, W2, b2, Ws, bs):
    n, d_in = x.shape
    e = edge_index.shape[1]
    assert d_in == 128

    npad = -(-(n + 1) // BR) * BR
    ept = -(-e // (NW * CH)) * CH
    epad = ept * NW
    nch = ept // CH

    src = edge_index[0]
    dst = edge_index[1]
    pad_e = epad - e
    src_p = jnp.concatenate([src, jnp.zeros((pad_e,), jnp.int32)])
    dst_p = jnp.concatenate([dst, jnp.full((pad_e,), n, jnp.int32)])
    x_p = jnp.zeros((npad, d_in), jnp.float32).at[:n].set(x)
    zeros1d = jnp.zeros((npad,), jnp.float32)
    zeros_row = jnp.zeros((npad, 128), jnp.float32)
    b0r = b0.reshape(1, 128)
    b1r = b1.reshape(1, 128)
    b2r = b2.reshape(1, 128)
    bsr = bs.reshape(1, 128)

    degp = _deg_partials(dst_p, zeros1d, npad, ept, nch)
    dinv = _dinv_from_deg(degp, npad)

    z0 = _z_proj(x_p, W0, dinv)
    s0 = _scatter_partials(z0, src_p, dst_p, zeros_row, npad, ept, nch)
    h0, z1 = _stage_b(s0, z0, dinv, b0r, W1)
    skip = _linear(h0, Ws, bsr)
    s1 = _scatter_partials(z1, src_p, dst_p, zeros_row, npad, ept, nch)
    z2 = _stage_c(s1, z1, dinv, b1r, skip, W2)
    s2 = _scatter_partials(z2, src_p, dst_p, zeros_row, npad, ept, nch)
    out = _stage_d(s2, z2, dinv, b2r)
    return out[:n]
